# Initial kernel scaffold; baseline (speedup 1.0000x reference)
import jax
import jax.numpy as jnp
from jax.experimental import pallas as pl


def kernel(x, token_table, pos_table):
    raise NotImplementedError("write your pallas kernel here")


# speedup vs baseline: 2.2068x; 2.2068x over previous
import functools

import jax
import jax.numpy as jnp
from jax import lax
from jax.experimental import pallas as pl
from jax.experimental.pallas import tpu as pltpu
from jax.experimental.pallas import tpu_sc as plsc

NC = 2
NS = 16
LANES = 16
NW = NC * NS

B, L, V, D = 4096, 200, 1000000, 64
N = B * L
N_PER_W = N // NW
CHUNK = 512
N_CHUNKS = N_PER_W // CHUNK
GATHER_SLICE = 128


def _body(x_hbm, tok_hbm, pos_hbm, out_hbm, idx_v, pidx_v, tokrows_v,
          posrows_v, sem):
  wid = lax.axis_index("s") * NC + lax.axis_index("c")

  def chunk_body(c, _):
    base = wid * N_PER_W + c * CHUNK
    pltpu.sync_copy(x_hbm.at[pl.ds(base, CHUNK)], idx_v)

    def pidx_body(g, _):
      xv = idx_v[pl.ds(g * LANES, LANES)]
      t = base + g * LANES + lax.iota(jnp.int32, LANES)
      p = lax.rem(t, L) + 1
      pidx_v[pl.ds(g * LANES, LANES)] = jnp.where(
          xv == 0, jnp.zeros((LANES,), jnp.int32), p)
      return 0
    lax.fori_loop(0, CHUNK // LANES, pidx_body, 0)

    copies = []
    for i in range(CHUNK // GATHER_SLICE):
      s = pl.ds(i * GATHER_SLICE, GATHER_SLICE)
      copies.append(pltpu.async_copy(
          tok_hbm.at[idx_v.at[s]], tokrows_v.at[s], sem))
      copies.append(pltpu.async_copy(
          pos_hbm.at[pidx_v.at[s]], posrows_v.at[s], sem))
    for cp in copies:
      cp.wait()

    def add_body(r, _):
      for j in range(D // LANES):
        s = pl.ds(j * LANES, LANES)
        tokrows_v[r, s] = tokrows_v[r, s] + posrows_v[r, s]
      return 0
    lax.fori_loop(0, CHUNK, add_body, 0)

    pltpu.sync_copy(tokrows_v, out_hbm.at[pl.ds(base, CHUNK)])
    return 0

  lax.fori_loop(0, N_CHUNKS, chunk_body, 0)


@jax.jit
def kernel(x, token_table, pos_table):
  kfn = pl.kernel(
      _body,
      out_type=jax.ShapeDtypeStruct((N, D), jnp.float32),
      mesh=plsc.VectorSubcoreMesh(core_axis_name="c", subcore_axis_name="s"),
      scratch_types=[
          pltpu.VMEM((CHUNK,), jnp.int32),
          pltpu.VMEM((CHUNK,), jnp.int32),
          pltpu.VMEM((CHUNK, D), jnp.float32),
          pltpu.VMEM((CHUNK, D), jnp.float32),
          pltpu.SemaphoreType.DMA,
      ],
      compiler_params=pltpu.CompilerParams(use_tc_tiling_on_sc=False),
  )
  out = kfn(x.reshape(N), token_table, pos_table)
  return out.reshape(B, L, D)
